# baseline (speedup 1.0000x reference)
import functools

import jax
import jax.numpy as jnp
from jax import lax
from jax.experimental import pallas as pl
from jax.experimental.pallas import tpu as pltpu
from jax.experimental.pallas import tpu_sc as plsc

N = 10000
D = 128
E = 320000

NC = 2
NS = 16
NW = NC * NS
EPW = E // NW
CHUNK = 80
NCHUNK = EPW // CHUNK
NP = 10112
ROWS_PT = NP // NS
ZCHUNKS = tuple((k * 80, 80) for k in range(7)) + ((560, 72),)

_mesh = plsc.VectorSubcoreMesh(core_axis_name="c", subcore_axis_name="s")


@functools.partial(
    pl.kernel,
    mesh=_mesh,
    out_type=jax.ShapeDtypeStruct((NC, NP, D), jnp.float32),
    scratch_types=[
        pltpu.VMEM((2, CHUNK), jnp.int32),
        pltpu.VMEM((CHUNK, D), jnp.float32),
        pltpu.VMEM_SHARED((NP, D), jnp.float32),
        pltpu.SemaphoreType.DMA,
        pltpu.SemaphoreType.DMA,
    ],
)
def _deg_kernel(dst_hbm, out_hbm, didx, ones_v, acc, zsem, isem):
    c = lax.axis_index("c")
    s = lax.axis_index("s")
    wid = c * NS + s
    ebase = wid * EPW

    def didx_src(ci):
        return dst_hbm.at[pl.ds(ebase + ci * CHUNK, CHUNK)]

    @pl.loop(0, CHUNK)
    def _(i):
        for j in range(D // 16):
            ones_v[i, pl.ds(j * 16, 16)] = jnp.zeros((16,), jnp.float32)

    base = s * ROWS_PT
    for off, ln in ZCHUNKS:
        pltpu.async_copy(ones_v.at[pl.ds(0, ln)], acc.at[pl.ds(base + off, ln)], zsem)
    for off, ln in ZCHUNKS:
        pltpu.make_async_copy(
            ones_v.at[pl.ds(0, ln)], acc.at[pl.ds(base + off, ln)], zsem).wait()

    @pl.loop(0, CHUNK)
    def _(i):
        for j in range(D // 16):
            ones_v[i, pl.ds(j * 16, 16)] = jnp.ones((16,), jnp.float32)
    plsc.subcore_barrier()

    pltpu.async_copy(didx_src(0), didx.at[0], isem)

    @pl.loop(0, NCHUNK)
    def _(ci):
        b = lax.rem(ci, 2)
        pltpu.make_async_copy(didx_src(ci), didx.at[b], isem).wait()

        @pl.when(ci + 1 < NCHUNK)
        def _():
            pltpu.async_copy(didx_src(ci + 1), didx.at[1 - b], isem)

        pltpu.sync_copy(ones_v, acc.at[didx.at[b]], add=True)

    plsc.subcore_barrier()
    sl = pl.ds(base, ROWS_PT)
    pltpu.sync_copy(acc.at[sl], out_hbm.at[c, sl])


@functools.partial(
    pl.kernel,
    mesh=_mesh,
    out_type=jax.ShapeDtypeStruct((NC, NP, D), jnp.float32),
    scratch_types=[
        pltpu.VMEM((2, CHUNK), jnp.int32),
        pltpu.VMEM((2, CHUNK), jnp.int32),
        pltpu.VMEM((2, CHUNK, D), jnp.float32),
        pltpu.VMEM_SHARED((NP, D), jnp.float32),
        pltpu.SemaphoreType.DMA,
        pltpu.SemaphoreType.DMA,
    ],
)
def _scatter_kernel(hs_hbm, src_hbm, dst_hbm, out_hbm,
                    sidx, didx, rows_v, acc, gsem, isem):
    c = lax.axis_index("c")
    s = lax.axis_index("s")
    wid = c * NS + s
    ebase = wid * EPW

    def sidx_src(ci):
        return src_hbm.at[pl.ds(ebase + ci * CHUNK, CHUNK)]

    def didx_src(ci):
        return dst_hbm.at[pl.ds(ebase + ci * CHUNK, CHUNK)]

    @pl.loop(0, CHUNK)
    def _(i):
        for j in range(D // 16):
            rows_v[0, i, pl.ds(j * 16, 16)] = jnp.zeros((16,), jnp.float32)

    base = s * ROWS_PT
    for off, ln in ZCHUNKS:
        pltpu.async_copy(rows_v.at[0, pl.ds(0, ln)], acc.at[pl.ds(base + off, ln)], gsem)
    for off, ln in ZCHUNKS:
        pltpu.make_async_copy(
            rows_v.at[0, pl.ds(0, ln)], acc.at[pl.ds(base + off, ln)], gsem).wait()
    plsc.subcore_barrier()

    pltpu.async_copy(sidx_src(0), sidx.at[0], isem)
    pltpu.async_copy(didx_src(0), didx.at[0], isem)
    pltpu.async_copy(sidx_src(1), sidx.at[1], isem)
    pltpu.async_copy(didx_src(1), didx.at[1], isem)
    pltpu.make_async_copy(sidx_src(0), sidx.at[0], isem).wait()
    pltpu.make_async_copy(didx_src(0), didx.at[0], isem).wait()
    pltpu.async_copy(hs_hbm.at[sidx.at[0]], rows_v.at[0], gsem)

    @pl.loop(0, NCHUNK)
    def _(ci):
        b = lax.rem(ci, 2)
        pltpu.make_async_copy(hs_hbm.at[sidx.at[b]], rows_v.at[b], gsem).wait()

        @pl.when(ci + 1 < NCHUNK)
        def _():
            pltpu.make_async_copy(
                sidx_src(ci + 1), sidx.at[1 - b], isem).wait()
            pltpu.make_async_copy(
                didx_src(ci + 1), didx.at[1 - b], isem).wait()
            pltpu.async_copy(hs_hbm.at[sidx.at[1 - b]], rows_v.at[1 - b], gsem)

        pltpu.sync_copy(rows_v.at[b], acc.at[didx.at[b]], add=True)

        @pl.when(ci + 2 < NCHUNK)
        def _():
            pltpu.async_copy(sidx_src(ci + 2), sidx.at[b], isem)
            pltpu.async_copy(didx_src(ci + 2), didx.at[b], isem)

    plsc.subcore_barrier()
    sl = pl.ds(base, ROWS_PT)
    pltpu.sync_copy(acc.at[sl], out_hbm.at[c, sl])


_BLK = 1000
_GRID = (N // _BLK,)


def _norm_from_deg(deg_ref):
    d = 1.0 + deg_ref[0, :, 0] + deg_ref[1, :, 0]
    return lax.rsqrt(d)[:, None]


def _mm1_body(x_ref, w_ref, hw_ref):
    hw_ref[...] = jnp.dot(x_ref[...], w_ref[...],
                          preferred_element_type=jnp.float32)


def _scale_body(deg_ref, hw_ref, hs_ref):
    hs_ref[...] = hw_ref[...] * _norm_from_deg(deg_ref)


def _mid_body(deg_ref, p_ref, hs_ref, b_ref, w_ref, o_ref):
    nrm = _norm_from_deg(deg_ref)
    agg = p_ref[0] + p_ref[1] + hs_ref[...]
    h = jnp.maximum(agg * nrm + b_ref[...], 0.0)
    o_ref[...] = jnp.dot(h, w_ref[...], preferred_element_type=jnp.float32) * nrm


def _fin_body(deg_ref, p_ref, hs_ref, b_ref, o_ref):
    nrm = _norm_from_deg(deg_ref)
    agg = p_ref[0] + p_ref[1] + hs_ref[...]
    o_ref[...] = agg * nrm + b_ref[...]


_deg_spec = pl.BlockSpec((NC, _BLK, D), lambda i: (0, i, 0))
_row_spec = pl.BlockSpec((_BLK, D), lambda i: (i, 0))
_p_spec = pl.BlockSpec((NC, _BLK, D), lambda i: (0, i, 0))
_w_spec = pl.BlockSpec((D, D), lambda i: (0, 0))
_b_spec = pl.BlockSpec((1, D), lambda i: (0, 0))
_out_t = jax.ShapeDtypeStruct((N, D), jnp.float32)

_mm1 = pl.pallas_call(
    _mm1_body, grid=_GRID,
    in_specs=[_row_spec, _w_spec],
    out_specs=_row_spec, out_shape=_out_t)

_scale = pl.pallas_call(
    _scale_body, grid=_GRID,
    in_specs=[_deg_spec, _row_spec],
    out_specs=_row_spec, out_shape=_out_t)

_mid = pl.pallas_call(
    _mid_body, grid=_GRID,
    in_specs=[_deg_spec, _p_spec, _row_spec, _b_spec, _w_spec],
    out_specs=_row_spec, out_shape=_out_t)

_fin = pl.pallas_call(
    _fin_body, grid=_GRID,
    in_specs=[_deg_spec, _p_spec, _row_spec, _b_spec],
    out_specs=_row_spec, out_shape=_out_t)


def kernel(x, edge_index, W1, b1, W2, b2, W3, b3):
    b1r = b1.reshape(1, D)
    b2r = b2.reshape(1, D)
    b3r = b3.reshape(1, D)

    src1 = edge_index[0]
    dst1 = edge_index[1]

    degp = _deg_kernel(dst1)
    hw1 = _mm1(x, W1)
    hs1 = _scale(degp, hw1)
    p1 = _scatter_kernel(hs1, src1, dst1)
    hs2 = _mid(degp, p1, hs1, b1r, W2)
    p2 = _scatter_kernel(hs2, src1, dst1)
    hs3 = _mid(degp, p2, hs2, b2r, W3)
    p3 = _scatter_kernel(hs3, src1, dst1)
    return _fin(degp, p3, hs3, b3r)


# speedup vs baseline: 1.0968x; 1.0968x over previous
import functools

import jax
import jax.numpy as jnp
from jax import lax
from jax.experimental import pallas as pl
from jax.experimental.pallas import tpu as pltpu
from jax.experimental.pallas import tpu_sc as plsc

N = 10000
D = 128
DH = D // 2
E = 320000

NC = 2
NS = 16
NW = NC * NS
EPW = E // NW
CHUNK = 125
SB = 16
NSB = EPW // (SB * CHUNK)
NCHUNK = NSB * SB
NP = 10112
ROWS_PT = NP // NS
ZCHUNKS = tuple((k * 80, 80) for k in range(7)) + ((560, 72),)

_mesh = plsc.VectorSubcoreMesh(core_axis_name="c", subcore_axis_name="s")


@functools.partial(
    pl.kernel,
    mesh=_mesh,
    out_type=jax.ShapeDtypeStruct((NC, NP, D), jnp.float32),
    scratch_types=[
        pltpu.VMEM((NSB, SB, CHUNK), jnp.int32),
        pltpu.VMEM((CHUNK, D), jnp.float32),
        pltpu.VMEM_SHARED((NP, D), jnp.float32),
        pltpu.SemaphoreType.DMA,
    ],
)
def _deg_kernel(dst_hbm, out_hbm, idx_v, ones_v, acc, zsem):
    c = lax.axis_index("c")
    s = lax.axis_index("s")
    wid = c * NS + s

    @pl.loop(0, CHUNK)
    def _(i):
        for j in range(D // 16):
            ones_v[i, pl.ds(j * 16, 16)] = jnp.zeros((16,), jnp.float32)

    base = s * ROWS_PT
    for off, ln in ZCHUNKS:
        pltpu.async_copy(ones_v.at[pl.ds(0, ln)], acc.at[pl.ds(base + off, ln)], zsem)
    for off, ln in ZCHUNKS:
        pltpu.make_async_copy(
            ones_v.at[pl.ds(0, ln)], acc.at[pl.ds(base + off, ln)], zsem).wait()

    @pl.loop(0, CHUNK)
    def _(i):
        for j in range(D // 16):
            ones_v[i, pl.ds(j * 16, 16)] = jnp.ones((16,), jnp.float32)
    plsc.subcore_barrier()

    pltpu.sync_copy(dst_hbm.at[wid], idx_v)

    @pl.loop(0, NSB)
    def _(sb):
        for k in range(SB):
            pltpu.sync_copy(ones_v, acc.at[idx_v.at[sb, k]], add=True)

    plsc.subcore_barrier()
    sl = pl.ds(base, ROWS_PT)
    pltpu.sync_copy(acc.at[sl], out_hbm.at[c, sl])


@functools.partial(
    pl.kernel,
    mesh=_mesh,
    out_type=jax.ShapeDtypeStruct((NC, NP, D), jnp.float32),
    scratch_types=[
        pltpu.VMEM((SB, CHUNK), jnp.int32),
        pltpu.VMEM((SB, CHUNK), jnp.int32),
        pltpu.VMEM((2, CHUNK, D), jnp.float32),
        pltpu.VMEM_SHARED((NP, D), jnp.float32),
        pltpu.SemaphoreType.DMA,
    ],
)
def _scatter_kernel(hs_hbm, src_hbm, dst_hbm, out_hbm,
                    sidx, didx, rowsf, acc, gsem):
    c = lax.axis_index("c")
    s = lax.axis_index("s")
    wid = c * NS + s

    @pl.loop(0, CHUNK)
    def _(i):
        for j in range(D // 16):
            rowsf[0, i, pl.ds(j * 16, 16)] = jnp.zeros((16,), jnp.float32)

    base = s * ROWS_PT
    for off, ln in ZCHUNKS:
        pltpu.async_copy(rowsf.at[0, pl.ds(0, ln)], acc.at[pl.ds(base + off, ln)], gsem)
    for off, ln in ZCHUNKS:
        pltpu.make_async_copy(
            rowsf.at[0, pl.ds(0, ln)], acc.at[pl.ds(base + off, ln)], gsem).wait()
    plsc.subcore_barrier()

    @pl.loop(0, NSB)
    def _(sb):
        pltpu.sync_copy(src_hbm.at[wid, sb], sidx)
        pltpu.sync_copy(dst_hbm.at[wid, sb], didx)
        pltpu.async_copy(hs_hbm.at[sidx.at[0]], rowsf.at[0], gsem)
        for k in range(SB):
            b = k % 2
            pltpu.make_async_copy(
                hs_hbm.at[sidx.at[k]], rowsf.at[b], gsem).wait()
            if k + 1 < SB:
                pltpu.async_copy(
                    hs_hbm.at[sidx.at[k + 1]], rowsf.at[1 - b], gsem)
            pltpu.sync_copy(rowsf.at[b], acc.at[didx.at[k]], add=True)

    plsc.subcore_barrier()
    sl = pl.ds(base, ROWS_PT)
    pltpu.sync_copy(acc.at[sl], out_hbm.at[c, sl])


_BLK = 1000
_GRID = (N // _BLK,)


def _norm_from_deg(deg_ref):
    d = 1.0 + deg_ref[0, :, 0] + deg_ref[1, :, 0]
    return lax.rsqrt(d)[:, None]


def _mm1_body(x_ref, w_ref, hw_ref):
    hw_ref[...] = jnp.dot(x_ref[...], w_ref[...],
                          preferred_element_type=jnp.float32)


def _scale_body(deg_ref, hw_ref, hs_ref):
    hs_ref[...] = hw_ref[...] * _norm_from_deg(deg_ref)


def _mid_body(deg_ref, p_ref, hs_ref, b_ref, w_ref, o_ref):
    nrm = _norm_from_deg(deg_ref)
    agg = p_ref[0] + p_ref[1] + hs_ref[...]
    h = jnp.maximum(agg * nrm + b_ref[...], 0.0)
    o_ref[...] = jnp.dot(h, w_ref[...], preferred_element_type=jnp.float32) * nrm


def _fin_body(deg_ref, p_ref, hs_ref, b_ref, o_ref):
    nrm = _norm_from_deg(deg_ref)
    agg = p_ref[0] + p_ref[1] + hs_ref[...]
    o_ref[...] = agg * nrm + b_ref[...]


_deg_spec = pl.BlockSpec((NC, _BLK, D), lambda i: (0, i, 0))
_row_spec = pl.BlockSpec((_BLK, D), lambda i: (i, 0))
_p_spec = pl.BlockSpec((NC, _BLK, D), lambda i: (0, i, 0))
_w_spec = pl.BlockSpec((D, D), lambda i: (0, 0))
_b_spec = pl.BlockSpec((1, D), lambda i: (0, 0))
_out_t = jax.ShapeDtypeStruct((N, D), jnp.float32)

_mm1 = pl.pallas_call(
    _mm1_body, grid=_GRID,
    in_specs=[_row_spec, _w_spec],
    out_specs=_row_spec, out_shape=_out_t)

_scale = pl.pallas_call(
    _scale_body, grid=_GRID,
    in_specs=[_deg_spec, _row_spec],
    out_specs=_row_spec, out_shape=_out_t)

_mid = pl.pallas_call(
    _mid_body, grid=_GRID,
    in_specs=[_deg_spec, _p_spec, _row_spec, _b_spec, _w_spec],
    out_specs=_row_spec, out_shape=_out_t)

_fin = pl.pallas_call(
    _fin_body, grid=_GRID,
    in_specs=[_deg_spec, _p_spec, _row_spec, _b_spec],
    out_specs=_row_spec, out_shape=_out_t)


def kernel(x, edge_index, W1, b1, W2, b2, W3, b3):
    src4 = edge_index[0].reshape(NW, NSB, SB, CHUNK)
    dst4 = edge_index[1].reshape(NW, NSB, SB, CHUNK)
    b1r = b1.reshape(1, D)
    b2r = b2.reshape(1, D)
    b3r = b3.reshape(1, D)

    degp = _deg_kernel(dst4)
    hw1 = _mm1(x, W1)
    hs1 = _scale(degp, hw1)
    p1 = _scatter_kernel(hs1, src4, dst4)
    hs2 = _mid(degp, p1, hs1, b1r, W2)
    p2 = _scatter_kernel(hs2, src4, dst4)
    hs3 = _mid(degp, p2, hs2, b2r, W3)
    p3 = _scatter_kernel(hs3, src4, dst4)
    return _fin(degp, p3, hs3, b3r)


# speedup vs baseline: 1.1013x; 1.0041x over previous
import functools

import jax
import jax.numpy as jnp
from jax import lax
from jax.experimental import pallas as pl
from jax.experimental.pallas import tpu as pltpu
from jax.experimental.pallas import tpu_sc as plsc

N = 10000
D = 128
E = 320000

NC = 2
NS = 16
NW = NC * NS
EPW = E // NW
CHUNK = 125
SB = 16
NSB = EPW // (SB * CHUNK)
NCHUNK = NSB * SB
NP = 10112
ROWS_PT = NP // NS
ZCHUNKS = tuple((k * 80, 80) for k in range(7)) + ((560, 72),)

_mesh = plsc.VectorSubcoreMesh(core_axis_name="c", subcore_axis_name="s")


@functools.partial(
    pl.kernel,
    mesh=_mesh,
    out_type=jax.ShapeDtypeStruct((NC, NP, D), jnp.float32),
    scratch_types=[
        pltpu.VMEM((NSB, SB, CHUNK), jnp.int32),
        pltpu.VMEM((CHUNK, D), jnp.float32),
        pltpu.VMEM_SHARED((NP, D), jnp.float32),
        pltpu.SemaphoreType.DMA,
    ],
)
def _deg_kernel(dst_hbm, out_hbm, idx_v, ones_v, acc, zsem):
    c = lax.axis_index("c")
    s = lax.axis_index("s")
    wid = c * NS + s

    @pl.loop(0, CHUNK)
    def _(i):
        for j in range(D // 16):
            ones_v[i, pl.ds(j * 16, 16)] = jnp.zeros((16,), jnp.float32)

    base = s * ROWS_PT
    for off, ln in ZCHUNKS:
        pltpu.async_copy(ones_v.at[pl.ds(0, ln)], acc.at[pl.ds(base + off, ln)], zsem)
    for off, ln in ZCHUNKS:
        pltpu.make_async_copy(
            ones_v.at[pl.ds(0, ln)], acc.at[pl.ds(base + off, ln)], zsem).wait()

    @pl.loop(0, CHUNK)
    def _(i):
        for j in range(D // 16):
            ones_v[i, pl.ds(j * 16, 16)] = jnp.ones((16,), jnp.float32)
    plsc.subcore_barrier()

    pltpu.sync_copy(dst_hbm.at[wid], idx_v)

    @pl.loop(0, NSB)
    def _(sb):
        for k in range(SB):
            pltpu.sync_copy(ones_v, acc.at[idx_v.at[sb, k]], add=True)

    plsc.subcore_barrier()
    sl = pl.ds(base, ROWS_PT)
    pltpu.sync_copy(acc.at[sl], out_hbm.at[c, sl])


@functools.partial(
    pl.kernel,
    mesh=_mesh,
    out_type=jax.ShapeDtypeStruct((NC, NP, D), jnp.float32),
    scratch_types=[
        pltpu.VMEM((SB, CHUNK), jnp.int32),
        pltpu.VMEM((SB, CHUNK), jnp.int32),
        pltpu.VMEM((2, CHUNK, D), jnp.float32),
        pltpu.VMEM_SHARED((NP, D), jnp.float32),
        pltpu.SemaphoreType.DMA,
    ],
)
def _scatter_kernel(hs_hbm, src_hbm, dst_hbm, out_hbm,
                    sidx, didx, rowsf, acc, gsem):
    c = lax.axis_index("c")
    s = lax.axis_index("s")
    wid = c * NS + s

    @pl.loop(0, CHUNK)
    def _(i):
        for j in range(D // 16):
            rowsf[0, i, pl.ds(j * 16, 16)] = jnp.zeros((16,), jnp.float32)

    base = s * ROWS_PT
    for off, ln in ZCHUNKS:
        pltpu.async_copy(rowsf.at[0, pl.ds(0, ln)], acc.at[pl.ds(base + off, ln)], gsem)
    for off, ln in ZCHUNKS:
        pltpu.make_async_copy(
            rowsf.at[0, pl.ds(0, ln)], acc.at[pl.ds(base + off, ln)], gsem).wait()
    plsc.subcore_barrier()

    @pl.loop(0, NSB)
    def _(sb):
        pltpu.sync_copy(src_hbm.at[wid, sb], sidx)
        pltpu.sync_copy(dst_hbm.at[wid, sb], didx)
        pltpu.async_copy(hs_hbm.at[sidx.at[0]], rowsf.at[0], gsem)
        for k in range(SB):
            b = k % 2
            pltpu.make_async_copy(
                hs_hbm.at[sidx.at[k]], rowsf.at[b], gsem).wait()
            if k + 1 < SB:
                pltpu.async_copy(
                    hs_hbm.at[sidx.at[k + 1]], rowsf.at[1 - b], gsem)
            pltpu.sync_copy(rowsf.at[b], acc.at[didx.at[k]], add=True)

    plsc.subcore_barrier()
    sl = pl.ds(base, ROWS_PT)
    pltpu.sync_copy(acc.at[sl], out_hbm.at[c, sl])


_BLK = 1000
_GRID = (N // _BLK,)


def _norm_from_deg(deg_ref):
    d = 1.0 + deg_ref[0, :, 0] + deg_ref[1, :, 0]
    return lax.rsqrt(d)[:, None]


def _mm1_body(x_ref, w_ref, hw_ref):
    hw_ref[...] = jnp.dot(x_ref[...], w_ref[...],
                          preferred_element_type=jnp.float32)


def _scale_body(deg_ref, hw_ref, hs_ref):
    hs_ref[...] = hw_ref[...] * _norm_from_deg(deg_ref)


def _mid_body(deg_ref, p_ref, hs_ref, b_ref, w_ref, o_ref):
    nrm = _norm_from_deg(deg_ref)
    agg = p_ref[0] + p_ref[1] + hs_ref[...]
    h = jnp.maximum(agg * nrm + b_ref[...], 0.0)
    o_ref[...] = jnp.dot(h, w_ref[...], preferred_element_type=jnp.float32) * nrm


def _fin_body(deg_ref, p_ref, hs_ref, b_ref, o_ref):
    nrm = _norm_from_deg(deg_ref)
    agg = p_ref[0] + p_ref[1] + hs_ref[...]
    o_ref[...] = agg * nrm + b_ref[...]


_deg_spec = pl.BlockSpec((NC, _BLK, D), lambda i: (0, i, 0))
_row_spec = pl.BlockSpec((_BLK, D), lambda i: (i, 0))
_p_spec = pl.BlockSpec((NC, _BLK, D), lambda i: (0, i, 0))
_w_spec = pl.BlockSpec((D, D), lambda i: (0, 0))
_b_spec = pl.BlockSpec((1, D), lambda i: (0, 0))
_out_t = jax.ShapeDtypeStruct((N, D), jnp.float32)

_mm1 = pl.pallas_call(
    _mm1_body, grid=_GRID,
    in_specs=[_row_spec, _w_spec],
    out_specs=_row_spec, out_shape=_out_t)

_scale = pl.pallas_call(
    _scale_body, grid=_GRID,
    in_specs=[_deg_spec, _row_spec],
    out_specs=_row_spec, out_shape=_out_t)

_mid = pl.pallas_call(
    _mid_body, grid=_GRID,
    in_specs=[_deg_spec, _p_spec, _row_spec, _b_spec, _w_spec],
    out_specs=_row_spec, out_shape=_out_t)

_fin = pl.pallas_call(
    _fin_body, grid=_GRID,
    in_specs=[_deg_spec, _p_spec, _row_spec, _b_spec],
    out_specs=_row_spec, out_shape=_out_t)


def kernel(x, edge_index, W1, b1, W2, b2, W3, b3):
    src4 = edge_index[0].reshape(NW, NSB, SB, CHUNK)
    dst4 = edge_index[1].reshape(NW, NSB, SB, CHUNK)
    b1r = b1.reshape(1, D)
    b2r = b2.reshape(1, D)
    b3r = b3.reshape(1, D)

    degp = _deg_kernel(dst4)
    hw1 = _mm1(x, W1)
    hs1 = _scale(degp, hw1)
    p1 = _scatter_kernel(hs1, src4, dst4)
    hs2 = _mid(degp, p1, hs1, b1r, W2)
    p2 = _scatter_kernel(hs2, src4, dst4)
    hs3 = _mid(degp, p2, hs2, b2r, W3)
    p3 = _scatter_kernel(hs3, src4, dst4)
    return _fin(degp, p3, hs3, b3r)
